# baseline (speedup 1.0000x reference)
import functools

import jax
import jax.numpy as jnp
from jax import lax
from jax.experimental import pallas as pl
from jax.experimental.pallas import tpu as pltpu
from jax.experimental.pallas import tpu_sc as plsc


def _make_sc_gather(B, D, CH):
    info = plsc.get_sparse_core_info()
    NC, NS = info.num_cores, info.num_subcores
    NW = NC * NS
    B2 = B // 2
    assert B2 % NW == 0
    r_per_w = B2 // NW
    assert r_per_w % CH == 0
    n_chunks = r_per_w // CH

    mesh = plsc.VectorSubcoreMesh(core_axis_name="c", subcore_axis_name="s")

    @functools.partial(
        pl.kernel,
        mesh=mesh,
        out_type=jax.ShapeDtypeStruct((B2, 2 * D), jnp.float32),
        scratch_types=[
            pltpu.VMEM((CH,), jnp.int32),
            pltpu.VMEM((CH,), jnp.int32),
            pltpu.VMEM((CH, D), jnp.float32),
            pltpu.VMEM((CH, D), jnp.float32),
            pltpu.SemaphoreType.DMA,
            pltpu.SemaphoreType.DMA,
            pltpu.SemaphoreType.DMA,
            pltpu.SemaphoreType.DMA,
        ],
        compiler_params=pltpu.CompilerParams(use_tc_tiling_on_sc=False),
    )
    def k(xe_hbm, xo_hbm, table_hbm, out128, i0, i1, r0, r1, sg0, sg1, sw0, sw1):
        idx_hbm = (xe_hbm, xo_hbm)
        idx_v = (i0, i1)
        rows_v = (r0, r1)
        sg = (sg0, sg1)
        sw = (sw0, sw1)
        wid = lax.axis_index("s") * NC + lax.axis_index("c")
        base = wid * r_per_w

        def out_dst(b, off):
            return out128.at[pl.ds(off, CH), b * D:(b + 1) * D]

        for b in range(2):
            pltpu.sync_copy(idx_hbm[b].at[pl.ds(base, CH)], idx_v[b])
            pltpu.async_copy(table_hbm.at[idx_v[b]], rows_v[b], sg[b])

        def step(g, b, prefetch):
            off = base + g * CH
            pltpu.make_async_copy(table_hbm.at[idx_v[b]], rows_v[b], sg[b]).wait()
            pltpu.async_copy(rows_v[b], out_dst(b, off), sw[b])
            if prefetch:
                pltpu.sync_copy(idx_hbm[b].at[pl.ds(off + CH, CH)], idx_v[b])
                pltpu.make_async_copy(rows_v[b], out_dst(b, off), sw[b]).wait()
                pltpu.async_copy(table_hbm.at[idx_v[b]], rows_v[b], sg[b])

        def outer(g, carry):
            for b in range(2):
                step(g, b, prefetch=True)
            return carry

        lax.fori_loop(0, n_chunks - 1, outer, 0)
        for b in range(2):
            g = n_chunks - 1
            step(g, b, prefetch=False)
            pltpu.make_async_copy(
                rows_v[b], out_dst(b, base + g * CH), sw[b]
            ).wait()

    return k


def kernel(x, table):
    Br, S = x.shape
    _, D = table.shape
    B = Br * S
    xf = x.reshape(B)
    out = _make_sc_gather(B, D, 800)(xf[0::2], xf[1::2], table)
    return out.reshape(Br, S, D)


# speedup vs baseline: 1.4315x; 1.4315x over previous
import functools

import jax
import jax.numpy as jnp
from jax import lax
from jax.experimental import pallas as pl
from jax.experimental.pallas import tpu as pltpu
from jax.experimental.pallas import tpu_sc as plsc


def _make_sc_gather(Br, S, D):
    info = plsc.get_sparse_core_info()
    NC, NS = info.num_cores, info.num_subcores
    NW = NC * NS
    NBT = Br // 128
    BT_W = NBT // NW
    BLK = 2 * 128
    NR = S * 8 * NBT * 8

    mesh = plsc.VectorSubcoreMesh(core_axis_name="c", subcore_axis_name="s")

    @functools.partial(
        pl.kernel,
        mesh=mesh,
        out_type=jax.ShapeDtypeStruct((NR, 128), jnp.float32),
        scratch_types=[
            pltpu.VMEM((BLK,), jnp.int32),
            pltpu.VMEM((BLK,), jnp.int32),
            pltpu.VMEM((BLK, D), jnp.float32),
            pltpu.VMEM((BLK, D), jnp.float32),
            pltpu.VMEM((D, 257), jnp.float32),
            pltpu.VMEM((D, 257), jnp.float32),
            pltpu.SemaphoreType.DMA,
            pltpu.SemaphoreType.DMA,
            pltpu.SemaphoreType.DMA,
            pltpu.SemaphoreType.DMA,
        ],
        compiler_params=pltpu.CompilerParams(
            use_tc_tiling_on_sc=False, needs_layout_passes=False
        ),
    )
    def k(xt_hbm, table_hbm, out3, i0, i1, r0, r1, t0, t1, sg0, sg1, sw0, sw1):
        idx_v = (i0, i1)
        rows_v = (r0, r1)
        rows_t = (t0, t1)
        sg = (sg0, sg1)
        sw = (sw0, sw1)
        wid = lax.axis_index("s") * NC + lax.axis_index("c")
        bt0 = wid * BT_W
        lanes = lax.broadcasted_iota(jnp.int32, (16,), 0)

        def idx_src(s, h):
            return xt_hbm.at[s, pl.ds((bt0 + 2 * h) * 128, BLK)]

        def fire_gather(s, h):
            pltpu.sync_copy(idx_src(s, h), idx_v[h])
            pltpu.async_copy(table_hbm.at[idx_v[h]], rows_v[h], sg[h])

        def fire_pieces(s, h):
            for btl in range(2):
                bt = bt0 + 2 * h + btl
                for dt in range(8):
                    r = (s * 8 + dt) * 128 + bt
                    pltpu.async_copy(
                        rows_t[h].at[pl.ds(dt * 8, 8), pl.ds(btl * 128, 128)],
                        out3.at[pl.ds(r * 8, 8), :],
                        sw[h],
                    )

        def wait_pieces(s, h):
            for btl in range(2):
                bt = bt0 + 2 * h + btl
                for dt in range(8):
                    r = (s * 8 + dt) * 128 + bt
                    pltpu.make_async_copy(
                        rows_t[h].at[pl.ds(dt * 8, 8), pl.ds(btl * 128, 128)],
                        out3.at[pl.ds(r * 8, 8), :],
                        sw[h],
                    ).wait()

        def transpose_block(h):
            def jbody(jj, carry):
                for u in range(4):
                    j = jj * 4 + u
                    jvec = jnp.broadcast_to(j, (16,)).astype(jnp.int32)
                    for c in range(4):
                        v = rows_v[h][j, pl.ds(c * 16, 16)]
                        plsc.store_scatter(
                            rows_t[h], [c * 16 + lanes, jvec], v
                        )
                return carry

            lax.fori_loop(0, BLK // 4, jbody, 0)

        def step(s, h, prefetch):
            pltpu.make_async_copy(table_hbm.at[idx_v[h]], rows_v[h], sg[h]).wait()
            transpose_block(h)
            fire_pieces(s, h)
            if prefetch:
                fire_gather(s + 1, h)
                wait_pieces(s, h)
            else:
                wait_pieces(s, h)

        for h in range(2):
            fire_gather(0, h)

        def outer(s, carry):
            for h in range(2):
                step(s, h, prefetch=True)
            return carry

        lax.fori_loop(0, S - 1, outer, 0)
        for h in range(2):
            step(S - 1, h, prefetch=False)

    return k


def kernel(x, table):
    Br, S = x.shape
    _, D = table.shape
    out3 = _make_sc_gather(Br, S, D)(x.T, table)
    out6 = out3.reshape(S, 8, Br // 128, 8, 128)
    return out6.transpose(2, 4, 0, 1, 3).reshape(Br, S, D)


# speedup vs baseline: 1.8823x; 1.3149x over previous
import functools

import jax
import jax.numpy as jnp
from jax import lax
from jax.experimental import pallas as pl
from jax.experimental.pallas import tpu as pltpu
from jax.experimental.pallas import tpu_sc as plsc


def _make_sc_gather(B, D, CH):
    info = plsc.get_sparse_core_info()
    NC, NS = info.num_cores, info.num_subcores
    NW = NC * NS
    assert B % NW == 0
    b_per_w = B // NW
    assert b_per_w % CH == 0
    n_chunks = b_per_w // CH

    mesh = plsc.VectorSubcoreMesh(core_axis_name="c", subcore_axis_name="s")

    @functools.partial(
        pl.kernel,
        mesh=mesh,
        out_type=jax.ShapeDtypeStruct((B, 2 * D), jnp.float32),
        scratch_types=[
            pltpu.VMEM((CH,), jnp.int32),
            pltpu.VMEM((CH,), jnp.int32),
            pltpu.VMEM((CH, D), jnp.float32),
            pltpu.VMEM((CH, D), jnp.float32),
            pltpu.SemaphoreType.DMA,
            pltpu.SemaphoreType.DMA,
            pltpu.SemaphoreType.DMA,
            pltpu.SemaphoreType.DMA,
            pltpu.SemaphoreType.DMA,
            pltpu.SemaphoreType.DMA,
        ],
        compiler_params=pltpu.CompilerParams(use_tc_tiling_on_sc=False),
    )
    def k(idx_hbm, table_hbm, out128, i0, i1, r0, r1, sg0, sg1, sw0, sw1, si0, si1):
        idx_v = (i0, i1)
        rows_v = (r0, r1)
        sg = (sg0, sg1)
        sw = (sw0, sw1)
        si = (si0, si1)
        wid = lax.axis_index("s") * NC + lax.axis_index("c")
        base = wid * b_per_w

        def out_dst(off):
            return out128.at[pl.ds(off, CH), 0:D]

        for b in range(2):
            off = base + b * CH
            pltpu.sync_copy(idx_hbm.at[pl.ds(off, CH)], idx_v[b])
            pltpu.async_copy(table_hbm.at[idx_v[b]], rows_v[b], sg[b])

        def step(i, b, prefetch):
            off = base + i * CH
            pltpu.make_async_copy(table_hbm.at[idx_v[b]], rows_v[b], sg[b]).wait()
            pltpu.async_copy(rows_v[b], out_dst(off), sw[b])
            if prefetch:
                pltpu.async_copy(
                    idx_hbm.at[pl.ds(off + 2 * CH, CH)], idx_v[b], si[b]
                )
                pltpu.make_async_copy(rows_v[b], out_dst(off), sw[b]).wait()
                pltpu.make_async_copy(
                    idx_hbm.at[pl.ds(off + 2 * CH, CH)], idx_v[b], si[b]
                ).wait()
                pltpu.async_copy(table_hbm.at[idx_v[b]], rows_v[b], sg[b])

        def outer(j, carry):
            for b in range(2):
                step(2 * j + b, b, prefetch=True)
            return carry

        lax.fori_loop(0, n_chunks // 2 - 1, outer, 0)
        for b in range(2):
            i = n_chunks - 2 + b
            step(i, b, prefetch=False)
            pltpu.make_async_copy(
                rows_v[b], out_dst(base + i * CH), sw[b]
            ).wait()

    return k


def kernel(x, table):
    Br, S = x.shape
    _, D = table.shape
    B = Br * S
    xf = x.reshape(B)
    out = _make_sc_gather(B, D, 800)(xf, table)
    return out[:, :D].reshape(Br, S, D)
